# baseline (speedup 1.0000x reference)
import functools

import jax
import jax.numpy as jnp
from jax.experimental import pallas as pl
from jax.experimental.pallas import tpu as pltpu


def _router_block(n_steps, target_load,
                  h_ref, m_ref, w_ref, mb_ref,
                  logits_ref, idx_ref, tkw_ref, aux_ref,
                  imp_ref):
    i = pl.program_id(0)
    E = w_ref.shape[0]
    blk = h_ref.shape[0]

    logits = jax.lax.dot_general(
        w_ref[...], h_ref[...],
        dimension_numbers=(((1,), (1,)), ((), ())),
        preferred_element_type=jnp.float32)
    logits = logits + mb_ref[...] * m_ref[...]
    logits_ref[...] = logits

    mx = jnp.max(logits, axis=0, keepdims=True)
    ex = jnp.exp(logits - mx)
    probs = ex / jnp.sum(ex, axis=0, keepdims=True)

    iota = jax.lax.broadcasted_iota(jnp.int32, probs.shape, 0)
    m1 = jnp.max(probs, axis=0, keepdims=True)
    i1 = jnp.min(jnp.where(probs == m1, iota, E), axis=0, keepdims=True)
    masked = jnp.where(iota == i1, -1.0, probs)
    m2 = jnp.max(masked, axis=0, keepdims=True)
    i2 = jnp.min(jnp.where(masked == m2, iota, E), axis=0, keepdims=True)

    tkw_ref[...] = jnp.concatenate([m1, m2], axis=0)
    idx_ref[...] = jnp.concatenate([i1, i2], axis=0)

    part = jnp.sum(probs, axis=1, keepdims=True)

    @pl.when(i == 0)
    def _():
        imp_ref[...] = part

    @pl.when(i > 0)
    def _():
        imp_ref[...] += part

    @pl.when(i == n_steps - 1)
    def _():
        diff = imp_ref[...] - target_load
        aux_ref[...] = jnp.mean(diff * diff, keepdims=True).reshape(1, 1)


def kernel(hidden_states, mass, W, mass_bias):
    B, T, C = hidden_states.shape
    E = W.shape[0]
    N = B * T
    BLK = 512
    n_steps = N // BLK
    target_load = float(N) / float(E)

    flat_h = hidden_states.reshape(N, C)
    m_row = mass.reshape(1, N)
    mb_col = mass_bias.reshape(E, 1)

    logits_t, idx_t, tkw_t, aux = pl.pallas_call(
        functools.partial(_router_block, n_steps, target_load),
        grid=(n_steps,),
        in_specs=[
            pl.BlockSpec((BLK, C), lambda i: (i, 0)),
            pl.BlockSpec((1, BLK), lambda i: (0, i)),
            pl.BlockSpec((E, C), lambda i: (0, 0)),
            pl.BlockSpec((E, 1), lambda i: (0, 0)),
        ],
        out_specs=[
            pl.BlockSpec((E, BLK), lambda i: (0, i)),
            pl.BlockSpec((2, BLK), lambda i: (0, i)),
            pl.BlockSpec((2, BLK), lambda i: (0, i)),
            pl.BlockSpec((1, 1), lambda i: (0, 0)),
        ],
        out_shape=[
            jax.ShapeDtypeStruct((E, N), jnp.float32),
            jax.ShapeDtypeStruct((2, N), jnp.int32),
            jax.ShapeDtypeStruct((2, N), jnp.float32),
            jax.ShapeDtypeStruct((1, 1), jnp.float32),
        ],
        scratch_shapes=[pltpu.VMEM((E, 1), jnp.float32)],
    )(flat_h, m_row, W, mb_col)

    return (logits_t.T, idx_t.T, aux.reshape(()), tkw_t.T)


# speedup vs baseline: 1.1558x; 1.1558x over previous
import functools

import jax
import jax.numpy as jnp
from jax.experimental import pallas as pl
from jax.experimental.pallas import tpu as pltpu


def _router_block(n_steps, target_load,
                  h_ref, m_ref, w_ref, mb_ref,
                  logits_ref, idx_ref, tkw_ref, aux_ref,
                  imp_ref):
    i = pl.program_id(0)
    E = w_ref.shape[0]
    blk = h_ref.shape[0]

    logits = jax.lax.dot_general(
        w_ref[...], h_ref[...],
        dimension_numbers=(((1,), (1,)), ((), ())),
        preferred_element_type=jnp.float32)
    logits = logits + mb_ref[...] * m_ref[...]
    logits_ref[...] = logits

    mx = jnp.max(logits, axis=0, keepdims=True)
    ex = jnp.exp(logits - mx)
    probs = ex / jnp.sum(ex, axis=0, keepdims=True)

    iota = jax.lax.broadcasted_iota(jnp.int32, probs.shape, 0)
    m1 = jnp.max(probs, axis=0, keepdims=True)
    i1 = jnp.min(jnp.where(probs == m1, iota, E), axis=0, keepdims=True)
    masked = jnp.where(iota == i1, -1.0, probs)
    m2 = jnp.max(masked, axis=0, keepdims=True)
    i2 = jnp.min(jnp.where(masked == m2, iota, E), axis=0, keepdims=True)

    tkw_ref[...] = jnp.concatenate([m1, m2], axis=0)
    idx_ref[...] = jnp.concatenate([i1, i2], axis=0)

    part = jnp.sum(probs, axis=1, keepdims=True)

    @pl.when(i == 0)
    def _():
        imp_ref[...] = part

    @pl.when(i > 0)
    def _():
        imp_ref[...] += part

    @pl.when(i == n_steps - 1)
    def _():
        diff = imp_ref[...] - target_load
        aux_ref[...] = jnp.mean(diff * diff, keepdims=True).reshape(1, 1)


def kernel(hidden_states, mass, W, mass_bias):
    B, T, C = hidden_states.shape
    E = W.shape[0]
    N = B * T
    BLK = 1024
    n_steps = N // BLK
    target_load = float(N) / float(E)

    flat_h = hidden_states.reshape(N, C)
    m_row = mass.reshape(1, N)
    mb_col = mass_bias.reshape(E, 1)

    logits_t, idx_t, tkw_t, aux = pl.pallas_call(
        functools.partial(_router_block, n_steps, target_load),
        grid=(n_steps,),
        in_specs=[
            pl.BlockSpec((BLK, C), lambda i: (i, 0)),
            pl.BlockSpec((1, BLK), lambda i: (0, i)),
            pl.BlockSpec((E, C), lambda i: (0, 0)),
            pl.BlockSpec((E, 1), lambda i: (0, 0)),
        ],
        out_specs=[
            pl.BlockSpec((E, BLK), lambda i: (0, i)),
            pl.BlockSpec((2, BLK), lambda i: (0, i)),
            pl.BlockSpec((2, BLK), lambda i: (0, i)),
            pl.BlockSpec((1, 1), lambda i: (0, 0)),
        ],
        out_shape=[
            jax.ShapeDtypeStruct((E, N), jnp.float32),
            jax.ShapeDtypeStruct((2, N), jnp.int32),
            jax.ShapeDtypeStruct((2, N), jnp.float32),
            jax.ShapeDtypeStruct((1, 1), jnp.float32),
        ],
        scratch_shapes=[pltpu.VMEM((E, 1), jnp.float32)],
    )(flat_h, m_row, W, mb_col)

    return (logits_t.T, idx_t.T, aux.reshape(()), tkw_t.T)
